# baseline (speedup 1.0000x reference)
import functools

import jax
import jax.numpy as jnp
import numpy as np
from jax import lax
from jax.experimental import pallas as pl
from jax.experimental.pallas import tpu as pltpu
from jax.experimental.pallas import tpu_sc as plsc

N = 10000
F = 128
E = 320000
E_H = E // 2
NTILES = 32
EPT = E_H // NTILES
CHUNK = 40
NCHUNK = EPT // CHUNK
NPAD = 10240
RPS = NPAD // 16
_EPS = 1e-5

BN = 2000
BE = 5000


def _gn(x, g, b):
    m = jnp.mean(x, axis=-1, keepdims=True)
    ms = jnp.mean(x * x, axis=-1, keepdims=True)
    v = ms - m * m
    return (x - m) * lax.rsqrt(v + _EPS) * g + b


def _bdot(a, b):
    return jnp.dot(a, b, preferred_element_type=jnp.float32)


def _pack_bf16(a, b):
    au = lax.bitcast_convert_type(a, jnp.uint32) + jnp.uint32(0x8000)
    bu = lax.bitcast_convert_type(b, jnp.uint32) + jnp.uint32(0x8000)
    w = (lax.shift_right_logical(au, jnp.uint32(16))
         | (bu & jnp.uint32(0xFFFF0000)))
    return lax.bitcast_convert_type(w, jnp.int32)


_PERM = np.stack([np.arange(F // 2), np.arange(F // 2) + F // 2], 1).reshape(-1)


def _node_pre_body(x_ref, wq, bq, gq, hq, wc1q, wc1w, wagt,
                   qc_ref, wc_ref, ob_ref):
    x = x_ref[...]
    q = _bdot(x, wq[...]) + bq[...]
    q = jnp.maximum(_gn(q, gq[...], hq[...]), 0.0)
    qcf = _bdot(q, wc1q[...])
    qc_ref[...] = _pack_bf16(qcf[:, :F // 2], qcf[:, F // 2:])
    wcf = _bdot(x, wc1w[...])
    wc_ref[...] = _pack_bf16(wcf[:, :F // 2], wcf[:, F // 2:])
    ob_ref[...] = _bdot(x, wagt[...])


_blk = lambda shape: pl.BlockSpec(shape, lambda i: (0, 0))
_row = lambda b: pl.BlockSpec((b, F), lambda i: (i, 0))

_node_pre = pl.pallas_call(
    _node_pre_body,
    grid=(N // BN,),
    in_specs=[_row(BN), _blk((F, F)), _blk((1, F)), _blk((1, F)), _blk((1, F)),
              _blk((F, F)), _blk((F, F)), _blk((F, F))],
    out_specs=[pl.BlockSpec((BN, F // 2), lambda i: (i, 0)),
               pl.BlockSpec((BN, F // 2), lambda i: (i, 0)), _row(BN)],
    out_shape=[jax.ShapeDtypeStruct((N, F // 2), jnp.int32),
               jax.ShapeDtypeStruct((N, F // 2), jnp.int32),
               jax.ShapeDtypeStruct((N, F), jnp.float32)],
)


def _edge_body(d2_ref, qg_ref, wg_ref, wd1, bd1, gd1, hd1, wd2, bd2, gd2, hd2,
               wc1d, bc1, gc1, hc1, wc2, c_ref):
    d2 = d2_ref[...]
    h = _bdot(d2, wd1[...]) + bd1[...]
    h = jnp.maximum(_gn(h, gd1[...], hd1[...]), 0.0)
    h = _bdot(h, wd2[...]) + bd2[...]
    h = jnp.maximum(_gn(h, gd2[...], hd2[...]), 0.0)
    e = (_bdot(h, wc1d[...]) + qg_ref[...].astype(jnp.float32)
         + wg_ref[...].astype(jnp.float32) + bc1[...])
    e = jnp.maximum(_gn(e, gc1[...], hc1[...]), 0.0)
    c_ref[...] = _bdot(e, wc2[...])


_edge_tc = pl.pallas_call(
    _edge_body,
    grid=(E_H // BE,),
    in_specs=[pl.BlockSpec((BE, 2), lambda i: (i, 0)), _row(BE), _row(BE),
              _blk((2, F)), _blk((1, F)), _blk((1, F)), _blk((1, F)),
              _blk((F, F)), _blk((1, F)), _blk((1, F)), _blk((1, F)),
              _blk((F, F)), _blk((1, F)), _blk((1, F)), _blk((1, F)),
              _blk((F, F))],
    out_specs=_row(BE),
    out_shape=jax.ShapeDtypeStruct((E_H, F), jnp.float32),
)


def _node_post_body(ob_ref, a0_ref, a1_ref, a2_ref, a3_ref, res_ref,
                    gng, gnb, wl, bl, gl, hl, out_ref):
    o = (ob_ref[...] + a0_ref[...] + a1_ref[...]
         + a2_ref[...] + a3_ref[...])
    o = jnp.maximum(_gn(o, gng[...], gnb[...]), 0.0)
    o = _bdot(o, wl[...]) + bl[...]
    o = _gn(o, gl[...], hl[...])
    out_ref[...] = jnp.maximum(o + res_ref[...], 0.0)


_node_post = pl.pallas_call(
    _node_post_body,
    grid=(N // BN,),
    in_specs=[_row(BN), _row(BN), _row(BN), _row(BN), _row(BN), _row(BN),
              _blk((1, F)), _blk((1, F)), _blk((F, F)), _blk((1, F)),
              _blk((1, F)), _blk((1, F))],
    out_specs=_row(BN),
    out_shape=jax.ShapeDtypeStruct((N, F), jnp.float32),
)


_sc_mesh = plsc.VectorSubcoreMesh(core_axis_name="c", subcore_axis_name="s")
_sc_params = pltpu.CompilerParams(needs_layout_passes=False,
                                  use_tc_tiling_on_sc=False)


def _gather_scratch(with_d2):
    s = []
    if with_d2:
        s += [pltpu.VMEM((2 * N,), jnp.float32),
              pltpu.VMEM((2 * EPT,), jnp.float32)]
    s += [pltpu.VMEM((EPT,), jnp.int32),
          pltpu.VMEM((EPT,), jnp.int32),
          pltpu.VMEM((CHUNK, F // 2), jnp.int32),
          pltpu.VMEM((CHUNK, F // 2), jnp.int32),
          pltpu.VMEM((CHUNK, F // 2), jnp.int32),
          pltpu.VMEM((CHUNK, F // 2), jnp.int32)]
    s += [pltpu.SemaphoreType.DMA] * 8
    return s


def _gather_pipeline(base, h1, w1, qc, wc, qg_out, wg_out, bq, bw, sgq, sgw,
                     swq, sww, mid_work):
    def gather(tbl, idx1, cj, buf, sem):
        pltpu.async_copy(tbl.at[idx1.at[pl.ds(cj * CHUNK, CHUNK)]], buf, sem)

    def wait_gather(tbl, idx1, cj, buf, sem):
        pltpu.make_async_copy(tbl.at[idx1.at[pl.ds(cj * CHUNK, CHUNK)]],
                              buf, sem).wait()

    def write(out, cj, buf, sem):
        pltpu.async_copy(buf, out.at[pl.ds(base + cj * CHUNK, CHUNK), :], sem)

    def wait_write(out, cj, buf, sem):
        pltpu.make_async_copy(buf, out.at[pl.ds(base + cj * CHUNK, CHUNK), :],
                              sem).wait()

    for b in range(2):
        gather(qc, h1, b, bq[b], sgq[b])
        gather(wc, w1, b, bw[b], sgw[b])

    mid_work()

    def pair(i, carry):
        j = i * 2
        for b in range(2):
            cj = j + b
            wait_gather(qc, h1, cj, bq[b], sgq[b])
            write(qg_out, cj, bq[b], swq[b])
            wait_gather(wc, w1, cj, bw[b], sgw[b])
            write(wg_out, cj, bw[b], sww[b])
        for b in range(2):
            cj = j + 2 + b

            @pl.when(cj < NCHUNK)
            def _issue(cj=cj, b=b):
                wait_write(qg_out, cj, bq[b], swq[b])
                gather(qc, h1, cj, bq[b], sgq[b])
                wait_write(wg_out, cj, bw[b], sww[b])
                gather(wc, w1, cj, bw[b], sgw[b])

        return carry

    lax.fori_loop(0, (NCHUNK - 1) // 2, pair, 0)

    last = NCHUNK - 1
    wait_gather(qc, h1, last, bq[0], sgq[0])
    write(qg_out, last, bq[0], swq[0])
    wait_gather(wc, w1, last, bw[0], sgw[0])
    write(wg_out, last, bw[0], sww[0])
    wait_write(qg_out, last, bq[0], swq[0])
    wait_write(wg_out, last, bw[0], sww[0])
    wait_write(qg_out, last - 1, bq[1], swq[1])
    wait_write(wg_out, last - 1, bw[1], sww[1])


@functools.partial(
    pl.kernel, mesh=_sc_mesh, compiler_params=_sc_params,
    out_type=[jax.ShapeDtypeStruct((2 * E_H,), jnp.float32),
              jax.ShapeDtypeStruct((E_H, F // 2), jnp.int32),
              jax.ShapeDtypeStruct((E_H, F // 2), jnp.int32)],
    scratch_types=_gather_scratch(True),
)
def _sc_gather_d2(hif, wif, ctrs, qc, wc, d2_out, qg_out, wg_out,
                  ctrs_v, d2_v, h1, w1, bq0, bq1, bw0, bw1,
                  sgq0, sgq1, sgw0, sgw1, swq0, swq1, sww0, sww1):
    w = lax.axis_index("c") * 16 + lax.axis_index("s")
    base = w * EPT
    pltpu.sync_copy(hif.at[pl.ds(base, EPT)], h1)
    pltpu.sync_copy(wif.at[pl.ds(base, EPT)], w1)

    def mid_work():
        pltpu.sync_copy(ctrs, ctrs_v)
        iota16 = lax.iota(jnp.int32, 16)
        one16 = jnp.full((16,), 1, jnp.int32)

        def d2_body(g, carry):
            h16 = 2 * h1[pl.ds(g * 16, 16)]
            w16 = 2 * w1[pl.ds(g * 16, 16)]
            xh = plsc.load_gather(ctrs_v, [h16])
            yh = plsc.load_gather(ctrs_v, [h16 + one16])
            xw = plsc.load_gather(ctrs_v, [w16])
            yw = plsc.load_gather(ctrs_v, [w16 + one16])
            r16 = 2 * (g * 16 + iota16)
            plsc.store_scatter(d2_v, [r16], xh - xw)
            plsc.store_scatter(d2_v, [r16 + one16], yh - yw)
            return carry

        lax.fori_loop(0, EPT // 16, d2_body, 0)
        pltpu.sync_copy(d2_v, d2_out.at[pl.ds(2 * base, 2 * EPT)])

    _gather_pipeline(base, h1, w1, qc, wc, qg_out, wg_out,
                     [bq0, bq1], [bw0, bw1], [sgq0, sgq1], [sgw0, sgw1],
                     [swq0, swq1], [sww0, sww1], mid_work)


@functools.partial(
    pl.kernel, mesh=_sc_mesh, compiler_params=_sc_params,
    out_type=[jax.ShapeDtypeStruct((E_H, F // 2), jnp.int32),
              jax.ShapeDtypeStruct((E_H, F // 2), jnp.int32)],
    scratch_types=_gather_scratch(False),
)
def _sc_gather_qw(hif, wif, qc, wc, qg_out, wg_out,
                  h1, w1, bq0, bq1, bw0, bw1,
                  sgq0, sgq1, sgw0, sgw1, swq0, swq1, sww0, sww1):
    w = lax.axis_index("c") * 16 + lax.axis_index("s")
    base = w * EPT
    pltpu.sync_copy(hif.at[pl.ds(base, EPT)], h1)
    pltpu.sync_copy(wif.at[pl.ds(base, EPT)], w1)
    _gather_pipeline(base, h1, w1, qc, wc, qg_out, wg_out,
                     [bq0, bq1], [bw0, bw1], [sgq0, sgq1], [sgw0, sgw1],
                     [swq0, swq1], [sww0, sww1], lambda: None)


@functools.partial(
    pl.kernel, mesh=_sc_mesh, compiler_params=_sc_params,
    out_type=jax.ShapeDtypeStruct((2, NPAD, F), jnp.float32),
    scratch_types=[pltpu.VMEM((NCHUNK, CHUNK), jnp.int32),
                   pltpu.VMEM((CHUNK, F), jnp.float32),
                   pltpu.VMEM((CHUNK, F), jnp.float32),
                   pltpu.VMEM_SHARED((NPAD, F), jnp.float32),
                   pltpu.SemaphoreType.DMA,
                   pltpu.SemaphoreType.DMA],
)
def _sc_scatter(hi3, c_in, zeros_nf, acc_out, h2, rb0, rb1, acc_sh, s0, s1):
    cid = lax.axis_index("c")
    sid = lax.axis_index("s")
    w = cid * 16 + sid
    bufs = [rb0, rb1]
    sems = [s0, s1]

    def load(cj, b):
        pltpu.async_copy(c_in.at[pl.ds(w * EPT + cj * CHUNK, CHUNK), :],
                         bufs[b], sems[b])

    def wait_load(cj, b):
        pltpu.make_async_copy(c_in.at[pl.ds(w * EPT + cj * CHUNK, CHUNK), :],
                              bufs[b], sems[b]).wait()

    load(0, 0)
    pltpu.sync_copy(zeros_nf.at[pl.ds(sid * RPS, RPS), :],
                    acc_sh.at[pl.ds(sid * RPS, RPS), :])
    pltpu.sync_copy(hi3.at[w], h2)
    plsc.subcore_barrier()

    def pair(i, carry):
        j = i * 2
        for b in range(2):
            cj = j + b

            @pl.when(cj + 1 < NCHUNK)
            def _prefetch(cj=cj, b=b):
                load(cj + 1, 1 - b)

            wait_load(cj, b)
            pltpu.sync_copy(bufs[b], acc_sh.at[h2.at[cj]], add=True)
        return carry

    lax.fori_loop(0, (NCHUNK - 1) // 2, pair, 0)
    last = NCHUNK - 1
    wait_load(last, 0)
    pltpu.sync_copy(bufs[0], acc_sh.at[h2.at[last]], add=True)
    plsc.subcore_barrier()
    pltpu.sync_copy(acc_sh.at[pl.ds(sid * RPS, RPS), :],
                    acc_out.at[cid, pl.ds(sid * RPS, RPS), :])


def kernel(actors, actor_ctrs, edge_index, params):
    hi = edge_index[0].astype(jnp.int32)
    wi = edge_index[1].astype(jnp.int32)
    hih = [hi[:E_H], hi[E_H:]]
    wih = [wi[:E_H], wi[E_H:]]
    hi3h = [h.reshape(NTILES, NCHUNK, CHUNK) for h in hih]
    ctrs = actor_ctrs.astype(jnp.float32).reshape(-1)
    zeros_nf = jnp.zeros((NPAD, F), jnp.float32)

    x = actors
    d2h = [None, None]
    for p in params:
        r = lambda v: v.reshape(1, F)
        qc, wcv, ob = _node_pre(x, p['Wq'], r(p['bq']), r(p['gq']),
                                r(p['hq']), p['Wc1'][F:2 * F],
                                p['Wc1'][2 * F:3 * F], p['Wagt'])
        gath = []
        for s in range(2):
            if d2h[s] is None:
                d2h[s], qg, wg = _sc_gather_d2(hih[s], wih[s], ctrs, qc, wcv)
            else:
                qg, wg = _sc_gather_qw(hih[s], wih[s], qc, wcv)
            gath.append((qg, wg))
        accs = []
        for s in range(2):
            qg, wg = gath[s]
            qgb = lax.bitcast_convert_type(qg, jnp.bfloat16).reshape(E_H, F)
            wgb = lax.bitcast_convert_type(wg, jnp.bfloat16).reshape(E_H, F)
            c = _edge_tc(d2h[s].reshape(E_H, 2), qgb, wgb,
                         p['Wd1'], r(p['bd1']), r(p['gd1']), r(p['hd1']),
                         p['Wd2'], r(p['bd2']), r(p['gd2']), r(p['hd2']),
                         p['Wc1'][0:F][:, _PERM], r(p['bc1'][_PERM]),
                         r(p['gc1'][_PERM]), r(p['hc1'][_PERM]),
                         p['Wc2'][_PERM, :])
            accs.append(_sc_scatter(hi3h[s], c, zeros_nf))
        x = _node_post(ob, accs[0][0], accs[0][1], accs[1][0], accs[1][1], x,
                       r(p['gn_g']), r(p['gn_b']),
                       p['Wl'], r(p['bl']), r(p['gl']), r(p['hl']))
    return x


# speedup vs baseline: 3.0645x; 3.0645x over previous
import functools

import jax
import jax.numpy as jnp
import numpy as np
from jax import lax
from jax.experimental import pallas as pl
from jax.experimental.pallas import tpu as pltpu
from jax.experimental.pallas import tpu_sc as plsc

N = 10000
F = 128
E = 320000
E_H = E // 2
NTILES = 32
EPT = E_H // NTILES
CHUNK = 40
NCHUNK = EPT // CHUNK
NPAD = 10240
RPS = NPAD // 16
_EPS = 1e-5

BN = 2000
BE = 5000


def _gn(x, g, b):
    m = jnp.mean(x, axis=-1, keepdims=True)
    ms = jnp.mean(x * x, axis=-1, keepdims=True)
    v = ms - m * m
    return (x - m) * lax.rsqrt(v + _EPS) * g + b


def _bdot(a, b):
    return jnp.dot(a, b, preferred_element_type=jnp.float32)


def _node_pre_body(x_ref, wq, bq, gq, hq, wc1q, wc1w, wagt,
                   qc_ref, wc_ref, ob_ref):
    x = x_ref[...]
    q = _bdot(x, wq[...]) + bq[...]
    q = jnp.maximum(_gn(q, gq[...], hq[...]), 0.0)
    qc_ref[...] = _bdot(q, wc1q[...])
    wc_ref[...] = _bdot(x, wc1w[...])
    ob_ref[...] = _bdot(x, wagt[...])


_blk = lambda shape: pl.BlockSpec(shape, lambda i: (0, 0))
_row = lambda b: pl.BlockSpec((b, F), lambda i: (i, 0))

_node_pre = pl.pallas_call(
    _node_pre_body,
    grid=(N // BN,),
    in_specs=[_row(BN), _blk((F, F)), _blk((1, F)), _blk((1, F)), _blk((1, F)),
              _blk((F, F)), _blk((F, F)), _blk((F, F))],
    out_specs=[_row(BN), _row(BN), _row(BN)],
    out_shape=[jax.ShapeDtypeStruct((N, F), jnp.float32)] * 3,
)


def _edge_body(d2_ref, qg_ref, wg_ref, wd1, bd1, gd1, hd1, wd2, bd2, gd2, hd2,
               wc1d, bc1, gc1, hc1, wc2, c_ref):
    d2 = d2_ref[...]
    h = _bdot(d2, wd1[...]) + bd1[...]
    h = jnp.maximum(_gn(h, gd1[...], hd1[...]), 0.0)
    h = _bdot(h, wd2[...]) + bd2[...]
    h = jnp.maximum(_gn(h, gd2[...], hd2[...]), 0.0)
    e = _bdot(h, wc1d[...]) + qg_ref[...] + wg_ref[...] + bc1[...]
    e = jnp.maximum(_gn(e, gc1[...], hc1[...]), 0.0)
    c_ref[...] = _bdot(e, wc2[...])


_edge_tc = pl.pallas_call(
    _edge_body,
    grid=(E_H // BE,),
    in_specs=[pl.BlockSpec((BE, 2), lambda i: (i, 0)), _row(BE), _row(BE),
              _blk((2, F)), _blk((1, F)), _blk((1, F)), _blk((1, F)),
              _blk((F, F)), _blk((1, F)), _blk((1, F)), _blk((1, F)),
              _blk((F, F)), _blk((1, F)), _blk((1, F)), _blk((1, F)),
              _blk((F, F))],
    out_specs=_row(BE),
    out_shape=jax.ShapeDtypeStruct((E_H, F), jnp.float32),
)


def _node_post_body(ob_ref, a0_ref, a1_ref, a2_ref, a3_ref, res_ref,
                    gng, gnb, wl, bl, gl, hl, out_ref):
    o = (ob_ref[...] + a0_ref[...] + a1_ref[...]
         + a2_ref[...] + a3_ref[...])
    o = jnp.maximum(_gn(o, gng[...], gnb[...]), 0.0)
    o = _bdot(o, wl[...]) + bl[...]
    o = _gn(o, gl[...], hl[...])
    out_ref[...] = jnp.maximum(o + res_ref[...], 0.0)


_node_post = pl.pallas_call(
    _node_post_body,
    grid=(N // BN,),
    in_specs=[_row(BN), _row(BN), _row(BN), _row(BN), _row(BN), _row(BN),
              _blk((1, F)), _blk((1, F)), _blk((F, F)), _blk((1, F)),
              _blk((1, F)), _blk((1, F))],
    out_specs=_row(BN),
    out_shape=jax.ShapeDtypeStruct((N, F), jnp.float32),
)


_sc_mesh = plsc.VectorSubcoreMesh(core_axis_name="c", subcore_axis_name="s")
_sc_params = pltpu.CompilerParams(needs_layout_passes=False)


def _gather_scratch(with_d2):
    s = []
    if with_d2:
        s += [pltpu.VMEM((2 * N,), jnp.float32),
              pltpu.VMEM((2 * EPT,), jnp.float32)]
    s += [pltpu.VMEM((EPT,), jnp.int32),
          pltpu.VMEM((EPT,), jnp.int32),
          pltpu.VMEM((CHUNK, F), jnp.float32),
          pltpu.VMEM((CHUNK, F), jnp.float32),
          pltpu.VMEM((CHUNK, F), jnp.float32),
          pltpu.VMEM((CHUNK, F), jnp.float32)]
    s += [pltpu.SemaphoreType.DMA] * 8
    return s


def _gather_pipeline(base, h1, w1, qc, wc, qg_out, wg_out, bq, bw, sgq, sgw,
                     swq, sww, mid_work):
    def gather(tbl, idx1, cj, buf, sem):
        pltpu.async_copy(tbl.at[idx1.at[pl.ds(cj * CHUNK, CHUNK)]], buf, sem)

    def wait_gather(tbl, idx1, cj, buf, sem):
        pltpu.make_async_copy(tbl.at[idx1.at[pl.ds(cj * CHUNK, CHUNK)]],
                              buf, sem).wait()

    def write(out, cj, buf, sem):
        pltpu.async_copy(buf, out.at[pl.ds(base + cj * CHUNK, CHUNK), :], sem)

    def wait_write(out, cj, buf, sem):
        pltpu.make_async_copy(buf, out.at[pl.ds(base + cj * CHUNK, CHUNK), :],
                              sem).wait()

    for b in range(2):
        gather(qc, h1, b, bq[b], sgq[b])
        gather(wc, w1, b, bw[b], sgw[b])

    mid_work()

    def pair(i, carry):
        j = i * 2
        for b in range(2):
            cj = j + b
            wait_gather(qc, h1, cj, bq[b], sgq[b])
            write(qg_out, cj, bq[b], swq[b])
            wait_gather(wc, w1, cj, bw[b], sgw[b])
            write(wg_out, cj, bw[b], sww[b])
        for b in range(2):
            cj = j + 2 + b

            @pl.when(cj < NCHUNK)
            def _issue(cj=cj, b=b):
                wait_write(qg_out, cj, bq[b], swq[b])
                gather(qc, h1, cj, bq[b], sgq[b])
                wait_write(wg_out, cj, bw[b], sww[b])
                gather(wc, w1, cj, bw[b], sgw[b])

        return carry

    lax.fori_loop(0, (NCHUNK - 1) // 2, pair, 0)

    last = NCHUNK - 1
    wait_gather(qc, h1, last, bq[0], sgq[0])
    write(qg_out, last, bq[0], swq[0])
    wait_gather(wc, w1, last, bw[0], sgw[0])
    write(wg_out, last, bw[0], sww[0])
    wait_write(qg_out, last, bq[0], swq[0])
    wait_write(wg_out, last, bw[0], sww[0])
    wait_write(qg_out, last - 1, bq[1], swq[1])
    wait_write(wg_out, last - 1, bw[1], sww[1])


@functools.partial(
    pl.kernel, mesh=_sc_mesh, compiler_params=_sc_params,
    out_type=[jax.ShapeDtypeStruct((2 * E_H,), jnp.float32),
              jax.ShapeDtypeStruct((E_H, F), jnp.float32),
              jax.ShapeDtypeStruct((E_H, F), jnp.float32)],
    scratch_types=_gather_scratch(True),
)
def _sc_gather_d2(hif, wif, ctrs, qc, wc, d2_out, qg_out, wg_out,
                  ctrs_v, d2_v, h1, w1, bq0, bq1, bw0, bw1,
                  sgq0, sgq1, sgw0, sgw1, swq0, swq1, sww0, sww1):
    w = lax.axis_index("c") * 16 + lax.axis_index("s")
    base = w * EPT
    pltpu.sync_copy(hif.at[pl.ds(base, EPT)], h1)
    pltpu.sync_copy(wif.at[pl.ds(base, EPT)], w1)

    def mid_work():
        pltpu.sync_copy(ctrs, ctrs_v)
        iota16 = lax.iota(jnp.int32, 16)
        one16 = jnp.full((16,), 1, jnp.int32)

        def d2_body(g, carry):
            h16 = 2 * h1[pl.ds(g * 16, 16)]
            w16 = 2 * w1[pl.ds(g * 16, 16)]
            xh = plsc.load_gather(ctrs_v, [h16])
            yh = plsc.load_gather(ctrs_v, [h16 + one16])
            xw = plsc.load_gather(ctrs_v, [w16])
            yw = plsc.load_gather(ctrs_v, [w16 + one16])
            r16 = 2 * (g * 16 + iota16)
            plsc.store_scatter(d2_v, [r16], xh - xw)
            plsc.store_scatter(d2_v, [r16 + one16], yh - yw)
            return carry

        lax.fori_loop(0, EPT // 16, d2_body, 0)
        pltpu.sync_copy(d2_v, d2_out.at[pl.ds(2 * base, 2 * EPT)])

    _gather_pipeline(base, h1, w1, qc, wc, qg_out, wg_out,
                     [bq0, bq1], [bw0, bw1], [sgq0, sgq1], [sgw0, sgw1],
                     [swq0, swq1], [sww0, sww1], mid_work)


@functools.partial(
    pl.kernel, mesh=_sc_mesh, compiler_params=_sc_params,
    out_type=[jax.ShapeDtypeStruct((E_H, F), jnp.float32),
              jax.ShapeDtypeStruct((E_H, F), jnp.float32)],
    scratch_types=_gather_scratch(False),
)
def _sc_gather_qw(hif, wif, qc, wc, qg_out, wg_out,
                  h1, w1, bq0, bq1, bw0, bw1,
                  sgq0, sgq1, sgw0, sgw1, swq0, swq1, sww0, sww1):
    w = lax.axis_index("c") * 16 + lax.axis_index("s")
    base = w * EPT
    pltpu.sync_copy(hif.at[pl.ds(base, EPT)], h1)
    pltpu.sync_copy(wif.at[pl.ds(base, EPT)], w1)
    _gather_pipeline(base, h1, w1, qc, wc, qg_out, wg_out,
                     [bq0, bq1], [bw0, bw1], [sgq0, sgq1], [sgw0, sgw1],
                     [swq0, swq1], [sww0, sww1], lambda: None)


@functools.partial(
    pl.kernel, mesh=_sc_mesh, compiler_params=_sc_params,
    out_type=jax.ShapeDtypeStruct((2, NPAD, F), jnp.float32),
    scratch_types=[pltpu.VMEM((NCHUNK, CHUNK), jnp.int32),
                   pltpu.VMEM((CHUNK, F), jnp.float32),
                   pltpu.VMEM((CHUNK, F), jnp.float32),
                   pltpu.VMEM_SHARED((NPAD, F), jnp.float32),
                   pltpu.SemaphoreType.DMA,
                   pltpu.SemaphoreType.DMA],
)
def _sc_scatter(hi3, c_in, zeros_nf, acc_out, h2, rb0, rb1, acc_sh, s0, s1):
    cid = lax.axis_index("c")
    sid = lax.axis_index("s")
    w = cid * 16 + sid
    bufs = [rb0, rb1]
    sems = [s0, s1]

    def load(cj, b):
        pltpu.async_copy(c_in.at[pl.ds(w * EPT + cj * CHUNK, CHUNK), :],
                         bufs[b], sems[b])

    def wait_load(cj, b):
        pltpu.make_async_copy(c_in.at[pl.ds(w * EPT + cj * CHUNK, CHUNK), :],
                              bufs[b], sems[b]).wait()

    load(0, 0)
    pltpu.sync_copy(zeros_nf.at[pl.ds(sid * RPS, RPS), :],
                    acc_sh.at[pl.ds(sid * RPS, RPS), :])
    pltpu.sync_copy(hi3.at[w], h2)
    plsc.subcore_barrier()

    def pair(i, carry):
        j = i * 2
        for b in range(2):
            cj = j + b

            @pl.when(cj + 1 < NCHUNK)
            def _prefetch(cj=cj, b=b):
                load(cj + 1, 1 - b)

            wait_load(cj, b)
            pltpu.sync_copy(bufs[b], acc_sh.at[h2.at[cj]], add=True)
        return carry

    lax.fori_loop(0, (NCHUNK - 1) // 2, pair, 0)
    last = NCHUNK - 1
    wait_load(last, 0)
    pltpu.sync_copy(bufs[0], acc_sh.at[h2.at[last]], add=True)
    plsc.subcore_barrier()
    pltpu.sync_copy(acc_sh.at[pl.ds(sid * RPS, RPS), :],
                    acc_out.at[cid, pl.ds(sid * RPS, RPS), :])


def kernel(actors, actor_ctrs, edge_index, params):
    hi = edge_index[0].astype(jnp.int32)
    wi = edge_index[1].astype(jnp.int32)
    hih = [hi[:E_H], hi[E_H:]]
    wih = [wi[:E_H], wi[E_H:]]
    hi3h = [h.reshape(NTILES, NCHUNK, CHUNK) for h in hih]
    ctrs = actor_ctrs.astype(jnp.float32).reshape(-1)
    zeros_nf = jnp.zeros((NPAD, F), jnp.float32)

    x = actors
    d2h = [None, None]
    for p in params:
        r = lambda v: v.reshape(1, F)
        qc, wcv, ob = _node_pre(x, p['Wq'], r(p['bq']), r(p['gq']),
                                r(p['hq']), p['Wc1'][F:2 * F],
                                p['Wc1'][2 * F:3 * F], p['Wagt'])
        gath = []
        for s in range(2):
            if d2h[s] is None:
                d2h[s], qg, wg = _sc_gather_d2(hih[s], wih[s], ctrs, qc, wcv)
            else:
                qg, wg = _sc_gather_qw(hih[s], wih[s], qc, wcv)
            gath.append((qg, wg))
        accs = []
        for s in range(2):
            qg, wg = gath[s]
            c = _edge_tc(d2h[s].reshape(E_H, 2), qg, wg,
                         p['Wd1'], r(p['bd1']), r(p['gd1']), r(p['hd1']),
                         p['Wd2'], r(p['bd2']), r(p['gd2']), r(p['hd2']),
                         p['Wc1'][0:F], r(p['bc1']), r(p['gc1']), r(p['hc1']),
                         p['Wc2'])
            accs.append(_sc_scatter(hi3h[s], c, zeros_nf))
        x = _node_post(ob, accs[0][0], accs[0][1], accs[1][0], accs[1][1], x,
                       r(p['gn_g']), r(p['gn_b']),
                       p['Wl'], r(p['bl']), r(p['gl']), r(p['hl']))
    return x


# speedup vs baseline: 3.0852x; 1.0068x over previous
import functools

import jax
import jax.numpy as jnp
import numpy as np
from jax import lax
from jax.experimental import pallas as pl
from jax.experimental.pallas import tpu as pltpu
from jax.experimental.pallas import tpu_sc as plsc

N = 10000
F = 128
E = 320000
E_H = E // 2
NTILES = 32
EPT = E_H // NTILES
CHUNK = 40
NCHUNK = EPT // CHUNK
GCHUNK = 128
GNCHUNK = EPT // GCHUNK
GTAIL = EPT - GNCHUNK * GCHUNK
NPAD = 10240
RPS = NPAD // 16
_EPS = 1e-5

BN = 2000
BE = 5000


def _gn(x, g, b):
    m = jnp.mean(x, axis=-1, keepdims=True)
    ms = jnp.mean(x * x, axis=-1, keepdims=True)
    v = ms - m * m
    return (x - m) * lax.rsqrt(v + _EPS) * g + b


def _bdot(a, b):
    return jnp.dot(a, b, preferred_element_type=jnp.float32)


def _node_pre_body(x_ref, wq, bq, gq, hq, wc1q, wc1w, wagt,
                   qc_ref, wc_ref, ob_ref):
    x = x_ref[...]
    q = _bdot(x, wq[...]) + bq[...]
    q = jnp.maximum(_gn(q, gq[...], hq[...]), 0.0)
    qc_ref[...] = _bdot(q, wc1q[...])
    wc_ref[...] = _bdot(x, wc1w[...])
    ob_ref[...] = _bdot(x, wagt[...])


_blk = lambda shape: pl.BlockSpec(shape, lambda i: (0, 0))
_row = lambda b: pl.BlockSpec((b, F), lambda i: (i, 0))

_node_pre = pl.pallas_call(
    _node_pre_body,
    grid=(N // BN,),
    in_specs=[_row(BN), _blk((F, F)), _blk((1, F)), _blk((1, F)), _blk((1, F)),
              _blk((F, F)), _blk((F, F)), _blk((F, F))],
    out_specs=[_row(BN), _row(BN), _row(BN)],
    out_shape=[jax.ShapeDtypeStruct((N, F), jnp.float32)] * 3,
)


def _edge_body(d2_ref, qg_ref, wg_ref, wd1, bd1, gd1, hd1, wd2, bd2, gd2, hd2,
               wc1d, bc1, gc1, hc1, wc2, c_ref):
    d2 = d2_ref[...]
    h = _bdot(d2, wd1[...]) + bd1[...]
    h = jnp.maximum(_gn(h, gd1[...], hd1[...]), 0.0)
    h = _bdot(h, wd2[...]) + bd2[...]
    h = jnp.maximum(_gn(h, gd2[...], hd2[...]), 0.0)
    e = _bdot(h, wc1d[...]) + qg_ref[...] + wg_ref[...] + bc1[...]
    e = jnp.maximum(_gn(e, gc1[...], hc1[...]), 0.0)
    c_ref[...] = _bdot(e, wc2[...])


_edge_tc = pl.pallas_call(
    _edge_body,
    grid=(E_H // BE,),
    in_specs=[pl.BlockSpec((BE, 2), lambda i: (i, 0)), _row(BE), _row(BE),
              _blk((2, F)), _blk((1, F)), _blk((1, F)), _blk((1, F)),
              _blk((F, F)), _blk((1, F)), _blk((1, F)), _blk((1, F)),
              _blk((F, F)), _blk((1, F)), _blk((1, F)), _blk((1, F)),
              _blk((F, F))],
    out_specs=_row(BE),
    out_shape=jax.ShapeDtypeStruct((E_H, F), jnp.float32),
)


def _node_post_body(ob_ref, a0_ref, a1_ref, a2_ref, a3_ref, res_ref,
                    gng, gnb, wl, bl, gl, hl, out_ref):
    o = (ob_ref[...] + a0_ref[...] + a1_ref[...]
         + a2_ref[...] + a3_ref[...])
    o = jnp.maximum(_gn(o, gng[...], gnb[...]), 0.0)
    o = _bdot(o, wl[...]) + bl[...]
    o = _gn(o, gl[...], hl[...])
    out_ref[...] = jnp.maximum(o + res_ref[...], 0.0)


_node_post = pl.pallas_call(
    _node_post_body,
    grid=(N // BN,),
    in_specs=[_row(BN), _row(BN), _row(BN), _row(BN), _row(BN), _row(BN),
              _blk((1, F)), _blk((1, F)), _blk((F, F)), _blk((1, F)),
              _blk((1, F)), _blk((1, F))],
    out_specs=_row(BN),
    out_shape=jax.ShapeDtypeStruct((N, F), jnp.float32),
)


_sc_mesh = plsc.VectorSubcoreMesh(core_axis_name="c", subcore_axis_name="s")
_sc_params = pltpu.CompilerParams(needs_layout_passes=False)


def _gather_scratch(with_d2):
    s = []
    if with_d2:
        s += [pltpu.VMEM((2 * N,), jnp.float32),
              pltpu.VMEM((2 * EPT,), jnp.float32)]
    s += [pltpu.VMEM((EPT,), jnp.int32),
          pltpu.VMEM((EPT,), jnp.int32),
          pltpu.VMEM((GCHUNK, F), jnp.float32),
          pltpu.VMEM((GCHUNK, F), jnp.float32),
          pltpu.VMEM((GCHUNK, F), jnp.float32),
          pltpu.VMEM((GCHUNK, F), jnp.float32)]
    s += [pltpu.SemaphoreType.DMA] * 8
    return s


def _gather_pipeline(base, h1, w1, qc, wc, qg_out, wg_out, bq, bw, sgq, sgw,
                     swq, sww, mid_work):
    def gather(tbl, idx1, cj, buf, sem):
        pltpu.async_copy(tbl.at[idx1.at[pl.ds(cj * GCHUNK, GCHUNK)]], buf, sem)

    def wait_gather(tbl, idx1, cj, buf, sem):
        pltpu.make_async_copy(tbl.at[idx1.at[pl.ds(cj * GCHUNK, GCHUNK)]],
                              buf, sem).wait()

    def write(out, cj, buf, sem):
        pltpu.async_copy(buf, out.at[pl.ds(base + cj * GCHUNK, GCHUNK), :],
                         sem)

    def wait_write(out, cj, buf, sem):
        pltpu.make_async_copy(buf,
                              out.at[pl.ds(base + cj * GCHUNK, GCHUNK), :],
                              sem).wait()

    for b in range(2):
        gather(qc, h1, b, bq[b], sgq[b])
        gather(wc, w1, b, bw[b], sgw[b])

    mid_work()

    def pair(i, carry):
        j = i * 2
        for b in range(2):
            cj = j + b
            wait_gather(qc, h1, cj, bq[b], sgq[b])
            write(qg_out, cj, bq[b], swq[b])
            wait_gather(wc, w1, cj, bw[b], sgw[b])
            write(wg_out, cj, bw[b], sww[b])
        for b in range(2):
            cj = j + 2 + b

            @pl.when(cj < GNCHUNK)
            def _issue(cj=cj, b=b):
                wait_write(qg_out, cj, bq[b], swq[b])
                gather(qc, h1, cj, bq[b], sgq[b])
                wait_write(wg_out, cj, bw[b], sww[b])
                gather(wc, w1, cj, bw[b], sgw[b])

        return carry

    lax.fori_loop(0, (GNCHUNK - 1) // 2, pair, 0)

    last = GNCHUNK - 1
    wait_gather(qc, h1, last, bq[0], sgq[0])
    write(qg_out, last, bq[0], swq[0])
    wait_gather(wc, w1, last, bw[0], sgw[0])
    write(wg_out, last, bw[0], sww[0])
    wait_write(qg_out, last, bq[0], swq[0])
    wait_write(wg_out, last, bw[0], sww[0])
    wait_write(qg_out, last - 1, bq[1], swq[1])
    wait_write(wg_out, last - 1, bw[1], sww[1])

    toff = EPT - GTAIL
    bqs = bq[0].at[pl.ds(0, GTAIL), :]
    bws = bw[0].at[pl.ds(0, GTAIL), :]
    pltpu.async_copy(qc.at[h1.at[pl.ds(toff, GTAIL)]], bqs, sgq[0])
    pltpu.async_copy(wc.at[w1.at[pl.ds(toff, GTAIL)]], bws, sgw[0])
    pltpu.make_async_copy(qc.at[h1.at[pl.ds(toff, GTAIL)]], bqs, sgq[0]).wait()
    pltpu.make_async_copy(wc.at[w1.at[pl.ds(toff, GTAIL)]], bws, sgw[0]).wait()
    pltpu.sync_copy(bqs, qg_out.at[pl.ds(base + toff, GTAIL), :])
    pltpu.sync_copy(bws, wg_out.at[pl.ds(base + toff, GTAIL), :])


@functools.partial(
    pl.kernel, mesh=_sc_mesh, compiler_params=_sc_params,
    out_type=[jax.ShapeDtypeStruct((2 * E_H,), jnp.float32),
              jax.ShapeDtypeStruct((E_H, F), jnp.float32),
              jax.ShapeDtypeStruct((E_H, F), jnp.float32)],
    scratch_types=_gather_scratch(True),
)
def _sc_gather_d2(hif, wif, ctrs, qc, wc, d2_out, qg_out, wg_out,
                  ctrs_v, d2_v, h1, w1, bq0, bq1, bw0, bw1,
                  sgq0, sgq1, sgw0, sgw1, swq0, swq1, sww0, sww1):
    w = lax.axis_index("c") * 16 + lax.axis_index("s")
    base = w * EPT
    pltpu.sync_copy(hif.at[pl.ds(base, EPT)], h1)
    pltpu.sync_copy(wif.at[pl.ds(base, EPT)], w1)

    def mid_work():
        pltpu.sync_copy(ctrs, ctrs_v)
        iota16 = lax.iota(jnp.int32, 16)
        one16 = jnp.full((16,), 1, jnp.int32)

        def d2_body(g, carry):
            h16 = 2 * h1[pl.ds(g * 16, 16)]
            w16 = 2 * w1[pl.ds(g * 16, 16)]
            xh = plsc.load_gather(ctrs_v, [h16])
            yh = plsc.load_gather(ctrs_v, [h16 + one16])
            xw = plsc.load_gather(ctrs_v, [w16])
            yw = plsc.load_gather(ctrs_v, [w16 + one16])
            r16 = 2 * (g * 16 + iota16)
            plsc.store_scatter(d2_v, [r16], xh - xw)
            plsc.store_scatter(d2_v, [r16 + one16], yh - yw)
            return carry

        lax.fori_loop(0, EPT // 16, d2_body, 0)
        pltpu.sync_copy(d2_v, d2_out.at[pl.ds(2 * base, 2 * EPT)])

    _gather_pipeline(base, h1, w1, qc, wc, qg_out, wg_out,
                     [bq0, bq1], [bw0, bw1], [sgq0, sgq1], [sgw0, sgw1],
                     [swq0, swq1], [sww0, sww1], mid_work)


@functools.partial(
    pl.kernel, mesh=_sc_mesh, compiler_params=_sc_params,
    out_type=[jax.ShapeDtypeStruct((E_H, F), jnp.float32),
              jax.ShapeDtypeStruct((E_H, F), jnp.float32)],
    scratch_types=_gather_scratch(False),
)
def _sc_gather_qw(hif, wif, qc, wc, qg_out, wg_out,
                  h1, w1, bq0, bq1, bw0, bw1,
                  sgq0, sgq1, sgw0, sgw1, swq0, swq1, sww0, sww1):
    w = lax.axis_index("c") * 16 + lax.axis_index("s")
    base = w * EPT
    pltpu.sync_copy(hif.at[pl.ds(base, EPT)], h1)
    pltpu.sync_copy(wif.at[pl.ds(base, EPT)], w1)
    _gather_pipeline(base, h1, w1, qc, wc, qg_out, wg_out,
                     [bq0, bq1], [bw0, bw1], [sgq0, sgq1], [sgw0, sgw1],
                     [swq0, swq1], [sww0, sww1], lambda: None)


@functools.partial(
    pl.kernel, mesh=_sc_mesh, compiler_params=_sc_params,
    out_type=jax.ShapeDtypeStruct((2, NPAD, F), jnp.float32),
    scratch_types=[pltpu.VMEM((NCHUNK, CHUNK), jnp.int32),
                   pltpu.VMEM((CHUNK, F), jnp.float32),
                   pltpu.VMEM((CHUNK, F), jnp.float32),
                   pltpu.VMEM_SHARED((NPAD, F), jnp.float32),
                   pltpu.SemaphoreType.DMA,
                   pltpu.SemaphoreType.DMA],
)
def _sc_scatter(hi3, c_in, zeros_nf, acc_out, h2, rb0, rb1, acc_sh, s0, s1):
    cid = lax.axis_index("c")
    sid = lax.axis_index("s")
    w = cid * 16 + sid
    bufs = [rb0, rb1]
    sems = [s0, s1]

    def load(cj, b):
        pltpu.async_copy(c_in.at[pl.ds(w * EPT + cj * CHUNK, CHUNK), :],
                         bufs[b], sems[b])

    def wait_load(cj, b):
        pltpu.make_async_copy(c_in.at[pl.ds(w * EPT + cj * CHUNK, CHUNK), :],
                              bufs[b], sems[b]).wait()

    load(0, 0)
    pltpu.sync_copy(zeros_nf.at[pl.ds(sid * RPS, RPS), :],
                    acc_sh.at[pl.ds(sid * RPS, RPS), :])
    pltpu.sync_copy(hi3.at[w], h2)
    plsc.subcore_barrier()

    def pair(i, carry):
        j = i * 2
        for b in range(2):
            cj = j + b

            @pl.when(cj + 1 < NCHUNK)
            def _prefetch(cj=cj, b=b):
                load(cj + 1, 1 - b)

            wait_load(cj, b)
            pltpu.sync_copy(bufs[b], acc_sh.at[h2.at[cj]], add=True)
        return carry

    lax.fori_loop(0, (NCHUNK - 1) // 2, pair, 0)
    last = NCHUNK - 1
    wait_load(last, 0)
    pltpu.sync_copy(bufs[0], acc_sh.at[h2.at[last]], add=True)
    plsc.subcore_barrier()
    pltpu.sync_copy(acc_sh.at[pl.ds(sid * RPS, RPS), :],
                    acc_out.at[cid, pl.ds(sid * RPS, RPS), :])


def kernel(actors, actor_ctrs, edge_index, params):
    hi = edge_index[0].astype(jnp.int32)
    wi = edge_index[1].astype(jnp.int32)
    hih = [hi[:E_H], hi[E_H:]]
    wih = [wi[:E_H], wi[E_H:]]
    hi3h = [h.reshape(NTILES, NCHUNK, CHUNK) for h in hih]
    ctrs = actor_ctrs.astype(jnp.float32).reshape(-1)
    zeros_nf = jnp.zeros((NPAD, F), jnp.float32)

    x = actors
    d2h = [None, None]
    for p in params:
        r = lambda v: v.reshape(1, F)
        qc, wcv, ob = _node_pre(x, p['Wq'], r(p['bq']), r(p['gq']),
                                r(p['hq']), p['Wc1'][F:2 * F],
                                p['Wc1'][2 * F:3 * F], p['Wagt'])
        gath = []
        for s in range(2):
            if d2h[s] is None:
                d2h[s], qg, wg = _sc_gather_d2(hih[s], wih[s], ctrs, qc, wcv)
            else:
                qg, wg = _sc_gather_qw(hih[s], wih[s], qc, wcv)
            gath.append((qg, wg))
        accs = []
        for s in range(2):
            qg, wg = gath[s]
            c = _edge_tc(d2h[s].reshape(E_H, 2), qg, wg,
                         p['Wd1'], r(p['bd1']), r(p['gd1']), r(p['hd1']),
                         p['Wd2'], r(p['bd2']), r(p['gd2']), r(p['hd2']),
                         p['Wc1'][0:F], r(p['bc1']), r(p['gc1']), r(p['hc1']),
                         p['Wc2'])
            accs.append(_sc_scatter(hi3h[s], c, zeros_nf))
        x = _node_post(ob, accs[0][0], accs[0][1], accs[1][0], accs[1][1], x,
                       r(p['gn_g']), r(p['gn_b']),
                       p['Wl'], r(p['bl']), r(p['gl']), r(p['hl']))
    return x


# speedup vs baseline: 3.0860x; 1.0003x over previous
import functools

import jax
import jax.numpy as jnp
from jax import lax
from jax.experimental import pallas as pl
from jax.experimental.pallas import tpu as pltpu
from jax.experimental.pallas import tpu_sc as plsc

N = 10000
F = 128
E = 320000
E_H = E // 2
NTILES = 32
EPT = E_H // NTILES
CHUNK = 40
NCHUNK = EPT // CHUNK
GCHUNK = 128
GNCHUNK = EPT // GCHUNK
GTAIL = EPT - GNCHUNK * GCHUNK
NPAD = 10240
RPS = NPAD // 16
_EPS = 1e-5

BN = 2000
BE = 5000


def _gn(x, g, b):
    m = jnp.mean(x, axis=-1, keepdims=True)
    ms = jnp.mean(x * x, axis=-1, keepdims=True)
    v = ms - m * m
    return (x - m) * lax.rsqrt(v + _EPS) * g + b


def _bdot(a, b):
    return jnp.dot(a, b, preferred_element_type=jnp.float32)


def _node_pre_body(x_ref, wq, bq, gq, hq, wc1q, wc1w, wagt,
                   qc_ref, wc_ref, ob_ref):
    x = x_ref[...]
    q = _bdot(x, wq[...]) + bq[...]
    q = jnp.maximum(_gn(q, gq[...], hq[...]), 0.0)
    qc_ref[...] = _bdot(q, wc1q[...])
    wc_ref[...] = _bdot(x, wc1w[...])
    ob_ref[...] = _bdot(x, wagt[...])


_blk = lambda shape: pl.BlockSpec(shape, lambda i: (0, 0))
_row = lambda b: pl.BlockSpec((b, F), lambda i: (i, 0))

_node_pre = pl.pallas_call(
    _node_pre_body,
    grid=(N // BN,),
    in_specs=[_row(BN), _blk((F, F)), _blk((1, F)), _blk((1, F)), _blk((1, F)),
              _blk((F, F)), _blk((F, F)), _blk((F, F))],
    out_specs=[_row(BN), _row(BN), _row(BN)],
    out_shape=[jax.ShapeDtypeStruct((N, F), jnp.float32)] * 3,
)


def _edge_body(d2_ref, qg_ref, wg_ref, wd1, bd1, gd1, hd1, wd2, bd2, gd2, hd2,
               wc1d, bc1, gc1, hc1, wc2, c_ref):
    d2 = d2_ref[...]
    h = _bdot(d2, wd1[...]) + bd1[...]
    h = jnp.maximum(_gn(h, gd1[...], hd1[...]), 0.0)
    h = _bdot(h, wd2[...]) + bd2[...]
    h = jnp.maximum(_gn(h, gd2[...], hd2[...]), 0.0)
    e = _bdot(h, wc1d[...]) + qg_ref[...] + wg_ref[...] + bc1[...]
    e = jnp.maximum(_gn(e, gc1[...], hc1[...]), 0.0)
    c_ref[...] = _bdot(e, wc2[...])


_edge_tc = pl.pallas_call(
    _edge_body,
    grid=(E_H // BE,),
    in_specs=[pl.BlockSpec((BE, 2), lambda i: (i, 0)), _row(BE), _row(BE),
              _blk((2, F)), _blk((1, F)), _blk((1, F)), _blk((1, F)),
              _blk((F, F)), _blk((1, F)), _blk((1, F)), _blk((1, F)),
              _blk((F, F)), _blk((1, F)), _blk((1, F)), _blk((1, F)),
              _blk((F, F))],
    out_specs=_row(BE),
    out_shape=jax.ShapeDtypeStruct((E_H, F), jnp.float32),
)


def _node_post_body(ob_ref, a0_ref, a1_ref, a2_ref, a3_ref, res_ref,
                    gng, gnb, wl, bl, gl, hl, out_ref):
    o = (ob_ref[...] + a0_ref[...] + a1_ref[...]
         + a2_ref[...] + a3_ref[...])
    o = jnp.maximum(_gn(o, gng[...], gnb[...]), 0.0)
    o = _bdot(o, wl[...]) + bl[...]
    o = _gn(o, gl[...], hl[...])
    out_ref[...] = jnp.maximum(o + res_ref[...], 0.0)


_node_post = pl.pallas_call(
    _node_post_body,
    grid=(N // BN,),
    in_specs=[_row(BN), _row(BN), _row(BN), _row(BN), _row(BN), _row(BN),
              _blk((1, F)), _blk((1, F)), _blk((F, F)), _blk((1, F)),
              _blk((1, F)), _blk((1, F))],
    out_specs=_row(BN),
    out_shape=jax.ShapeDtypeStruct((N, F), jnp.float32),
)


_sc_mesh = plsc.VectorSubcoreMesh(core_axis_name="c", subcore_axis_name="s")
_sc_params = pltpu.CompilerParams(needs_layout_passes=False)


def _gather_scratch(with_d2):
    s = []
    if with_d2:
        s += [pltpu.VMEM((2 * N,), jnp.float32),
              pltpu.VMEM((2 * EPT,), jnp.float32)]
    s += [pltpu.VMEM((EPT,), jnp.int32),
          pltpu.VMEM((EPT,), jnp.int32),
          pltpu.VMEM((GCHUNK, F), jnp.float32),
          pltpu.VMEM((GCHUNK, F), jnp.float32),
          pltpu.VMEM((GCHUNK, F), jnp.float32),
          pltpu.VMEM((GCHUNK, F), jnp.float32)]
    s += [pltpu.SemaphoreType.DMA] * 8
    return s


def _gather_pipeline(base, h1, w1, qc, wc, qg_out, wg_out, bq, bw, sgq, sgw,
                     swq, sww, mid_work):
    def gather(tbl, idx1, cj, buf, sem):
        pltpu.async_copy(tbl.at[idx1.at[pl.ds(cj * GCHUNK, GCHUNK)]], buf, sem)

    def wait_gather(tbl, idx1, cj, buf, sem):
        pltpu.make_async_copy(tbl.at[idx1.at[pl.ds(cj * GCHUNK, GCHUNK)]],
                              buf, sem).wait()

    def write(out, cj, buf, sem):
        pltpu.async_copy(buf, out.at[pl.ds(base + cj * GCHUNK, GCHUNK), :],
                         sem)

    def wait_write(out, cj, buf, sem):
        pltpu.make_async_copy(buf,
                              out.at[pl.ds(base + cj * GCHUNK, GCHUNK), :],
                              sem).wait()

    for b in range(2):
        gather(qc, h1, b, bq[b], sgq[b])
        gather(wc, w1, b, bw[b], sgw[b])

    mid_work()

    def pair(i, carry):
        j = i * 2
        for b in range(2):
            cj = j + b
            wait_gather(qc, h1, cj, bq[b], sgq[b])
            write(qg_out, cj, bq[b], swq[b])
            wait_gather(wc, w1, cj, bw[b], sgw[b])
            write(wg_out, cj, bw[b], sww[b])
        for b in range(2):
            cj = j + 2 + b

            @pl.when(cj < GNCHUNK)
            def _issue(cj=cj, b=b):
                wait_write(qg_out, cj, bq[b], swq[b])
                gather(qc, h1, cj, bq[b], sgq[b])
                wait_write(wg_out, cj, bw[b], sww[b])
                gather(wc, w1, cj, bw[b], sgw[b])

        return carry

    lax.fori_loop(0, (GNCHUNK - 1) // 2, pair, 0)

    last = GNCHUNK - 1
    wait_gather(qc, h1, last, bq[0], sgq[0])
    write(qg_out, last, bq[0], swq[0])
    wait_gather(wc, w1, last, bw[0], sgw[0])
    write(wg_out, last, bw[0], sww[0])
    wait_write(qg_out, last, bq[0], swq[0])
    wait_write(wg_out, last, bw[0], sww[0])
    wait_write(qg_out, last - 1, bq[1], swq[1])
    wait_write(wg_out, last - 1, bw[1], sww[1])

    toff = EPT - GTAIL
    bqs = bq[0].at[pl.ds(0, GTAIL), :]
    bws = bw[0].at[pl.ds(0, GTAIL), :]
    pltpu.async_copy(qc.at[h1.at[pl.ds(toff, GTAIL)]], bqs, sgq[0])
    pltpu.async_copy(wc.at[w1.at[pl.ds(toff, GTAIL)]], bws, sgw[0])
    pltpu.make_async_copy(qc.at[h1.at[pl.ds(toff, GTAIL)]], bqs, sgq[0]).wait()
    pltpu.make_async_copy(wc.at[w1.at[pl.ds(toff, GTAIL)]], bws, sgw[0]).wait()
    pltpu.sync_copy(bqs, qg_out.at[pl.ds(base + toff, GTAIL), :])
    pltpu.sync_copy(bws, wg_out.at[pl.ds(base + toff, GTAIL), :])


@functools.partial(
    pl.kernel, mesh=_sc_mesh, compiler_params=_sc_params,
    out_type=[jax.ShapeDtypeStruct((2 * E_H,), jnp.float32),
              jax.ShapeDtypeStruct((E_H, F), jnp.float32),
              jax.ShapeDtypeStruct((E_H, F), jnp.float32)],
    scratch_types=_gather_scratch(True),
)
def _sc_gather_d2(hif, wif, ctrs, qc, wc, d2_out, qg_out, wg_out,
                  ctrs_v, d2_v, h1, w1, bq0, bq1, bw0, bw1,
                  sgq0, sgq1, sgw0, sgw1, swq0, swq1, sww0, sww1):
    w = lax.axis_index("c") * 16 + lax.axis_index("s")
    base = w * EPT
    pltpu.sync_copy(hif.at[pl.ds(base, EPT)], h1)
    pltpu.sync_copy(wif.at[pl.ds(base, EPT)], w1)

    def mid_work():
        pltpu.sync_copy(ctrs, ctrs_v)
        iota16 = lax.iota(jnp.int32, 16)
        one16 = jnp.full((16,), 1, jnp.int32)

        def d2_body(g, carry):
            h16 = 2 * h1[pl.ds(g * 16, 16)]
            w16 = 2 * w1[pl.ds(g * 16, 16)]
            xh = plsc.load_gather(ctrs_v, [h16])
            yh = plsc.load_gather(ctrs_v, [h16 + one16])
            xw = plsc.load_gather(ctrs_v, [w16])
            yw = plsc.load_gather(ctrs_v, [w16 + one16])
            r16 = 2 * (g * 16 + iota16)
            plsc.store_scatter(d2_v, [r16], xh - xw)
            plsc.store_scatter(d2_v, [r16 + one16], yh - yw)
            return carry

        lax.fori_loop(0, EPT // 16, d2_body, 0)
        pltpu.sync_copy(d2_v, d2_out.at[pl.ds(2 * base, 2 * EPT)])

    _gather_pipeline(base, h1, w1, qc, wc, qg_out, wg_out,
                     [bq0, bq1], [bw0, bw1], [sgq0, sgq1], [sgw0, sgw1],
                     [swq0, swq1], [sww0, sww1], mid_work)


@functools.partial(
    pl.kernel, mesh=_sc_mesh, compiler_params=_sc_params,
    out_type=[jax.ShapeDtypeStruct((E_H, F), jnp.float32),
              jax.ShapeDtypeStruct((E_H, F), jnp.float32)],
    scratch_types=_gather_scratch(False),
)
def _sc_gather_qw(hif, wif, qc, wc, qg_out, wg_out,
                  h1, w1, bq0, bq1, bw0, bw1,
                  sgq0, sgq1, sgw0, sgw1, swq0, swq1, sww0, sww1):
    w = lax.axis_index("c") * 16 + lax.axis_index("s")
    base = w * EPT
    pltpu.sync_copy(hif.at[pl.ds(base, EPT)], h1)
    pltpu.sync_copy(wif.at[pl.ds(base, EPT)], w1)
    _gather_pipeline(base, h1, w1, qc, wc, qg_out, wg_out,
                     [bq0, bq1], [bw0, bw1], [sgq0, sgq1], [sgw0, sgw1],
                     [swq0, swq1], [sww0, sww1], lambda: None)


@functools.partial(
    pl.kernel, mesh=_sc_mesh, compiler_params=_sc_params,
    out_type=jax.ShapeDtypeStruct((2, NPAD, F), jnp.float32),
    scratch_types=[pltpu.VMEM((NCHUNK, CHUNK), jnp.int32),
                   pltpu.VMEM((CHUNK, F), jnp.float32),
                   pltpu.VMEM((CHUNK, F), jnp.float32),
                   pltpu.VMEM_SHARED((NPAD, F), jnp.float32),
                   pltpu.SemaphoreType.DMA,
                   pltpu.SemaphoreType.DMA],
)
def _sc_scatter(hi3, c_in, zeros_nf, acc_out, h2, rb0, rb1, acc_sh, s0, s1):
    cid = lax.axis_index("c")
    sid = lax.axis_index("s")
    w = cid * 16 + sid
    bufs = [rb0, rb1]
    sems = [s0, s1]

    def load(cj, b):
        pltpu.async_copy(c_in.at[pl.ds(w * EPT + cj * CHUNK, CHUNK), :],
                         bufs[b], sems[b])

    def wait_load(cj, b):
        pltpu.make_async_copy(c_in.at[pl.ds(w * EPT + cj * CHUNK, CHUNK), :],
                              bufs[b], sems[b]).wait()

    load(0, 0)
    pltpu.sync_copy(zeros_nf.at[pl.ds(sid * RPS, RPS), :],
                    acc_sh.at[pl.ds(sid * RPS, RPS), :])
    pltpu.sync_copy(hi3.at[w], h2)
    plsc.subcore_barrier()

    def pair(i, carry):
        j = i * 2
        for b in range(2):
            cj = j + b

            @pl.when(cj + 1 < NCHUNK)
            def _prefetch(cj=cj, b=b):
                load(cj + 1, 1 - b)

            wait_load(cj, b)
            pltpu.sync_copy(bufs[b], acc_sh.at[h2.at[cj]], add=True)
        return carry

    lax.fori_loop(0, (NCHUNK - 1) // 2, pair, 0)
    last = NCHUNK - 1
    wait_load(last, 0)
    pltpu.sync_copy(bufs[0], acc_sh.at[h2.at[last]], add=True)
    plsc.subcore_barrier()
    pltpu.sync_copy(acc_sh.at[pl.ds(sid * RPS, RPS), :],
                    acc_out.at[cid, pl.ds(sid * RPS, RPS), :])


def kernel(actors, actor_ctrs, edge_index, params):
    hi = edge_index[0].astype(jnp.int32)
    wi = edge_index[1].astype(jnp.int32)
    hih = [hi[:E_H], hi[E_H:]]
    wih = [wi[:E_H], wi[E_H:]]
    hi3h = [h.reshape(NTILES, NCHUNK, CHUNK) for h in hih]
    ctrs = actor_ctrs.astype(jnp.float32).reshape(-1)
    zeros_nf = jnp.zeros((NPAD, F), jnp.float32)

    x = actors
    d2h = [None, None]
    for p in params:
        r = lambda v: v.reshape(1, F)
        qc, wcv, ob = _node_pre(x, p['Wq'], r(p['bq']), r(p['gq']),
                                r(p['hq']), p['Wc1'][F:2 * F],
                                p['Wc1'][2 * F:3 * F], p['Wagt'])
        gath = []
        for s in range(2):
            if d2h[s] is None:
                d2h[s], qg, wg = _sc_gather_d2(hih[s], wih[s], ctrs, qc, wcv)
            else:
                qg, wg = _sc_gather_qw(hih[s], wih[s], qc, wcv)
            gath.append((qg, wg))
        accs = []
        for s in range(2):
            qg, wg = gath[s]
            c = _edge_tc(d2h[s].reshape(E_H, 2), qg, wg,
                         p['Wd1'], r(p['bd1']), r(p['gd1']), r(p['hd1']),
                         p['Wd2'], r(p['bd2']), r(p['gd2']), r(p['hd2']),
                         p['Wc1'][0:F], r(p['bc1']), r(p['gc1']), r(p['hc1']),
                         p['Wc2'])
            accs.append(_sc_scatter(hi3h[s], c, zeros_nf))
        x = _node_post(ob, accs[0][0], accs[0][1], accs[1][0], accs[1][1], x,
                       r(p['gn_g']), r(p['gn_b']),
                       p['Wl'], r(p['bl']), r(p['gl']), r(p['hl']))
    return x
